# Initial kernel scaffold; baseline (speedup 1.0000x reference)
import functools

import jax
import jax.numpy as jnp
from jax import lax
from jax.experimental import pallas as pl
from jax.experimental.pallas import tpu as pltpu
from jax.experimental.pallas import tpu_sc as plsc

N = 10000
E = 320000
D = 128
K = 10
ALPHA = 0.1

NC = 2
NS = 16
L = 16
DH = D // 2

CHUNK = 128
EP_TILE = -(-E // (NS * CHUNK)) * CHUNK
NCHUNK = EP_TILE // CHUNK
E_PAD = EP_TILE * NS

UB = 125
NUPD = N // NS // UB


def _sc_body(x0_hbm, colg, rowg, valg, xout,
             cols_v, rows_v, vals_v, gbuf, ubuf, hbuf, zbuf, acc):
    c = lax.axis_index("c")
    s = lax.axis_index("s")
    base_rows = s * (N // NS)

    pltpu.sync_copy(colg.at[s], cols_v)
    pltpu.sync_copy(rowg.at[s], rows_v)
    pltpu.sync_copy(valg.at[s], vals_v)

    coff = c * N

    def _adj(j, _):
        for g in range(CHUNK // L):
            sl = pl.ds(g * L, L)
            cols_v[j, sl] = cols_v[j, sl] + coff
        return 0

    lax.fori_loop(0, NCHUNK, _adj, 0)

    def _zero(i, _):
        for g in range(DH // L):
            zbuf[i, pl.ds(g * L, L)] = jnp.zeros((L,), jnp.float32)
        return 0

    lax.fori_loop(0, UB, _zero, 0)

    def _init(u, _):
        b = coff + base_rows + u * UB
        pltpu.sync_copy(x0_hbm.at[pl.ds(b, UB)], ubuf)
        pltpu.sync_copy(ubuf, xout.at[pl.ds(b, UB)])
        pltpu.sync_copy(zbuf, acc.at[pl.ds(base_rows + u * UB, UB)])
        return 0

    lax.fori_loop(0, NUPD, _init, 0)
    plsc.subcore_barrier()

    def _step(_, carry):
        def _chunk(j, _c):
            pltpu.sync_copy(xout.at[cols_v.at[j]], gbuf)

            def _scale(e, _e):
                v = vals_v[j, e]
                for g in range(DH // L):
                    sl = pl.ds(g * L, L)
                    gbuf[e, sl] = gbuf[e, sl] * v
                return 0

            lax.fori_loop(0, CHUNK, _scale, 0)
            pltpu.sync_copy(gbuf, acc.at[rows_v.at[j]], add=True)
            return 0

        lax.fori_loop(0, NCHUNK, _chunk, 0)
        plsc.subcore_barrier()

        def _upd(u, _u):
            b = base_rows + u * UB
            pltpu.sync_copy(acc.at[pl.ds(b, UB)], ubuf)
            pltpu.sync_copy(x0_hbm.at[pl.ds(coff + b, UB)], hbuf)

            def _mix(i, _i):
                for g in range(DH // L):
                    sl = pl.ds(g * L, L)
                    ubuf[i, sl] = (1.0 - ALPHA) * ubuf[i, sl] \
                        + ALPHA * hbuf[i, sl]
                return 0

            lax.fori_loop(0, UB, _mix, 0)
            pltpu.sync_copy(ubuf, xout.at[pl.ds(coff + b, UB)])
            pltpu.sync_copy(zbuf, acc.at[pl.ds(b, UB)])
            return 0

        lax.fori_loop(0, NUPD, _upd, 0)
        plsc.subcore_barrier()
        return carry

    lax.fori_loop(0, K, _step, 0)


@jax.jit
def kernel(x, adj_indices, adj_values):
    row = adj_indices[0].astype(jnp.int32)
    col = adj_indices[1].astype(jnp.int32)
    val = adj_values.astype(jnp.float32)

    pad = E_PAD - E
    row = jnp.concatenate([row, jnp.zeros((pad,), jnp.int32)])
    col = jnp.concatenate([col, jnp.zeros((pad,), jnp.int32)])
    val = jnp.concatenate([val, jnp.zeros((pad,), jnp.float32)])

    colg = col.reshape(NS, NCHUNK, CHUNK)
    rowg = row.reshape(NS, NCHUNK, CHUNK)
    valg = val.reshape(NS, NCHUNK, CHUNK)

    x0 = x.reshape(N, NC, DH).transpose(1, 0, 2).reshape(NC * N, DH)

    mesh = plsc.VectorSubcoreMesh(core_axis_name="c", subcore_axis_name="s")
    xout = pl.kernel(
        _sc_body,
        out_type=jax.ShapeDtypeStruct((NC * N, DH), jnp.float32),
        mesh=mesh,
        scratch_types=[
            pltpu.VMEM((NCHUNK, CHUNK), jnp.int32),
            pltpu.VMEM((NCHUNK, CHUNK), jnp.int32),
            pltpu.VMEM((NCHUNK, CHUNK), jnp.float32),
            pltpu.VMEM((CHUNK, DH), jnp.float32),
            pltpu.VMEM((UB, DH), jnp.float32),
            pltpu.VMEM((UB, DH), jnp.float32),
            pltpu.VMEM((UB, DH), jnp.float32),
            pltpu.VMEM_SHARED((N, DH), jnp.float32),
        ],
    )(x0, colg, rowg, valg)

    return xout.reshape(NC, N, DH).transpose(1, 0, 2).reshape(N, D)


# speedup vs baseline: 2.0571x; 2.0571x over previous
import jax
import jax.numpy as jnp
from jax import lax
from jax.experimental import pallas as pl
from jax.experimental.pallas import tpu as pltpu
from jax.experimental.pallas import tpu_sc as plsc

N = 10000
E = 320000
D = 128
K = 10
ALPHA = 0.1

NS = 16
L = 16

CHUNK = 128
SB = 8
GRP = SB * CHUNK
EP_TILE = -(-E // (NS * GRP)) * GRP
NG = EP_TILE // GRP
E_PAD = EP_TILE * NS

NP2 = 10240
NT = NP2 // NS
UB = 32
NUPD = NT // UB


def _sc_body(x0_hbm, colg, rowg, valg, xout,
             cbuf, rbuf, vbuf, gbuf, ubuf, hbuf, zbuf, acc):
    s = lax.axis_index("s")
    base_rows = s * NT

    def _zero(i, _):
        for g in range(D // L):
            zbuf[i, pl.ds(g * L, L)] = jnp.zeros((L,), jnp.float32)
        return 0

    lax.fori_loop(0, UB, _zero, 0)

    def _init(u, _):
        b = base_rows + u * UB
        pltpu.sync_copy(x0_hbm.at[pl.ds(b, UB)], ubuf)
        pltpu.sync_copy(ubuf, xout.at[pl.ds(b, UB)])
        pltpu.sync_copy(zbuf, acc.at[pl.ds(b, UB)])
        return 0

    lax.fori_loop(0, NUPD, _init, 0)
    plsc.subcore_barrier()

    def _step(_, carry):
        def _group(g, _g):
            pltpu.sync_copy(colg.at[s, g], cbuf)
            pltpu.sync_copy(rowg.at[s, g], rbuf)
            pltpu.sync_copy(valg.at[s, g], vbuf)

            def _chunk(j, _c):
                pltpu.sync_copy(xout.at[cbuf.at[j]], gbuf)

                def _scale(q, _e):
                    vv = vbuf[j, pl.ds(q * L, L)]
                    for i in range(L):
                        v = vv[i]
                        e = q * L + i
                        for f in range(D // L):
                            sl = pl.ds(f * L, L)
                            gbuf[e, sl] = gbuf[e, sl] * v
                    return 0

                lax.fori_loop(0, CHUNK // L, _scale, 0)
                pltpu.sync_copy(gbuf, acc.at[rbuf.at[j]], add=True)
                return 0

            lax.fori_loop(0, SB, _chunk, 0)
            return 0

        lax.fori_loop(0, NG, _group, 0)
        plsc.subcore_barrier()

        def _upd(u, _u):
            b = base_rows + u * UB
            pltpu.sync_copy(acc.at[pl.ds(b, UB)], ubuf)
            pltpu.sync_copy(x0_hbm.at[pl.ds(b, UB)], hbuf)

            def _mix(i, _i):
                for g in range(D // L):
                    sl = pl.ds(g * L, L)
                    ubuf[i, sl] = (1.0 - ALPHA) * ubuf[i, sl] \
                        + ALPHA * hbuf[i, sl]
                return 0

            lax.fori_loop(0, UB, _mix, 0)
            pltpu.sync_copy(ubuf, xout.at[pl.ds(b, UB)])
            pltpu.sync_copy(zbuf, acc.at[pl.ds(b, UB)])
            return 0

        lax.fori_loop(0, NUPD, _upd, 0)
        plsc.subcore_barrier()
        return carry

    lax.fori_loop(0, K, _step, 0)


@jax.jit
def kernel(x, adj_indices, adj_values):
    row = adj_indices[0].astype(jnp.int32)
    col = adj_indices[1].astype(jnp.int32)
    val = adj_values.astype(jnp.float32)

    pad = E_PAD - E
    row = jnp.concatenate([row, jnp.zeros((pad,), jnp.int32)])
    col = jnp.concatenate([col, jnp.zeros((pad,), jnp.int32)])
    val = jnp.concatenate([val, jnp.zeros((pad,), jnp.float32)])

    colg = col.reshape(NS, NG, SB, CHUNK)
    rowg = row.reshape(NS, NG, SB, CHUNK)
    valg = val.reshape(NS, NG, SB, CHUNK)

    x0 = jnp.pad(x, ((0, NP2 - N), (0, 0)))

    mesh = plsc.VectorSubcoreMesh(
        core_axis_name="c", subcore_axis_name="s", num_cores=1)
    xout = pl.kernel(
        _sc_body,
        out_type=jax.ShapeDtypeStruct((NP2, D), jnp.float32),
        mesh=mesh,
        scratch_types=[
            pltpu.VMEM((SB, CHUNK), jnp.int32),
            pltpu.VMEM((SB, CHUNK), jnp.int32),
            pltpu.VMEM((SB, CHUNK), jnp.float32),
            pltpu.VMEM((CHUNK, D), jnp.float32),
            pltpu.VMEM((UB, D), jnp.float32),
            pltpu.VMEM((UB, D), jnp.float32),
            pltpu.VMEM((UB, D), jnp.float32),
            pltpu.VMEM_SHARED((NP2, D), jnp.float32),
        ],
    )(x0, colg, rowg, valg)

    return xout[:N]
